# baseline (speedup 1.0000x reference)
import functools

import jax
import jax.numpy as jnp
from jax import lax
from jax.experimental import pallas as pl
from jax.experimental.pallas import tpu as pltpu
from jax.experimental.pallas import tpu_sc as plsc

B = 1024
L = 200
D = 64
DPAD = 128
VOCAB = 100000

NUM_CORES = 2
NUM_SUBCORES = 16
NW = NUM_CORES * NUM_SUBCORES
SEQ_W = B // NW
HALF0 = 128
HALF1 = L - HALF0
NCHUNK = 2 * SEQ_W
NBUF = 4
LOOKAHEAD = 2

_mesh = plsc.VectorSubcoreMesh(core_axis_name="c", subcore_axis_name="s")


@functools.partial(
    pl.kernel,
    out_type=jax.ShapeDtypeStruct((B, L, DPAD), jnp.float32),
    mesh=_mesh,
    scratch_types=[
        pltpu.VMEM((SEQ_W, HALF0), jnp.int32),
        pltpu.VMEM((SEQ_W, HALF1), jnp.int32),
        pltpu.VMEM((NBUF, HALF0, D), jnp.float32),
        pltpu.VMEM((L, D), jnp.float32),
        pltpu.SemaphoreType.DMA((NBUF,)),
        pltpu.SemaphoreType.DMA((NBUF,)),
    ],
    compiler_params=pltpu.CompilerParams(use_tc_tiling_on_sc=False),
)
def _embed_kernel(xa_hbm, xb_hbm, tok_hbm, pos_hbm, out_hbm,
                  idxA, idxB, rows, pos_v, sem_g, sem_o):
    wid = lax.axis_index("s") * NUM_CORES + lax.axis_index("c")
    sbase = wid * SEQ_W

    pltpu.sync_copy(xa_hbm.at[pl.ds(sbase, SEQ_W)], idxA)
    pltpu.sync_copy(xb_hbm.at[pl.ds(sbase, SEQ_W)], idxB)
    pltpu.sync_copy(pos_hbm, pos_v)

    def halves(k, b):
        h = b & 1
        off = HALF0 * h
        n = HALF1 if h else HALF0
        return k >> 1, off, n

    def idx_ref(s, b):
        if b & 1:
            return idxB.at[s, pl.ds(0, HALF1)]
        return idxA.at[s, pl.ds(0, HALF0)]

    def g_issue(k, b):
        s, _, n = halves(k, b)
        pltpu.async_copy(
            tok_hbm.at[idx_ref(s, b)], rows.at[b, pl.ds(0, n)], sem_g.at[b])

    def g_wait(k, b):
        s, _, n = halves(k, b)
        pltpu.make_async_copy(
            tok_hbm.at[idx_ref(s, b)], rows.at[b, pl.ds(0, n)],
            sem_g.at[b]).wait()

    def o_copy(k, b):
        s, off, n = halves(k, b)
        return pltpu.make_async_copy(
            rows.at[b, pl.ds(0, n)],
            out_hbm.at[sbase + s, pl.ds(off, n), pl.ds(0, D)],
            sem_o.at[b])

    def chunk_step(k, b, issue_next, out_wait):
        g_wait(k, b)
        b2 = (b + LOOKAHEAD) % NBUF
        if out_wait:
            o_copy(k - (NBUF - LOOKAHEAD), b2).wait()
        if issue_next:
            g_issue(k + LOOKAHEAD, b2)
        _, off, n = halves(k, b)
        rows_b = rows.at[b]

        @plsc.parallel_loop(0, n, unroll=4)
        def _row(r):
            for c in range(D // 16):
                sl = pl.ds(c * 16, 16)
                rows_b[r, sl] = rows_b[r, sl] + pos_v[off + r, sl]

        o_copy(k, b).start()

    for j in range(LOOKAHEAD):
        g_issue(j, j)
    for k in range(NBUF):
        chunk_step(k, k, True, k >= NBUF - LOOKAHEAD)

    @pl.loop(1, NCHUNK // NBUF - 1)
    def _group(g):
        k0 = g * NBUF
        for b in range(NBUF):
            chunk_step(k0 + b, b, True, True)

    for k in range(NCHUNK - NBUF, NCHUNK):
        chunk_step(k, k % NBUF, k + LOOKAHEAD < NCHUNK, True)
    for k in range(NCHUNK - NBUF + LOOKAHEAD, NCHUNK):
        o_copy(k, k % NBUF).wait()


def kernel(x, token_table, position_table):
    out = _embed_kernel(x[:, :HALF0], x[:, HALF0:], token_table,
                        position_table)
    return out[:, :, :D]


# speedup vs baseline: 1.0012x; 1.0012x over previous
import functools

import jax
import jax.numpy as jnp
from jax import lax
from jax.experimental import pallas as pl
from jax.experimental.pallas import tpu as pltpu
from jax.experimental.pallas import tpu_sc as plsc

B = 1024
L = 200
D = 64
DPAD = 128
VOCAB = 100000

NUM_CORES = 2
NUM_SUBCORES = 16
NW = NUM_CORES * NUM_SUBCORES
SEQ_W = B // NW
HALF0 = 128
HALF1 = L - HALF0
XB_W = 80
XB_OFF = L - XB_W
NCHUNK = 2 * SEQ_W
NBUF = 4
LOOKAHEAD = 2

_mesh = plsc.VectorSubcoreMesh(core_axis_name="c", subcore_axis_name="s")


@functools.partial(
    pl.kernel,
    out_type=jax.ShapeDtypeStruct((B, L, DPAD), jnp.float32),
    mesh=_mesh,
    scratch_types=[
        pltpu.VMEM((SEQ_W, HALF0), jnp.float32),
        pltpu.VMEM((SEQ_W, XB_W), jnp.float32),
        pltpu.VMEM((SEQ_W, HALF0), jnp.int32),
        pltpu.VMEM((SEQ_W, XB_W), jnp.int32),
        pltpu.VMEM((NBUF, HALF0, D), jnp.float32),
        pltpu.VMEM((L, D), jnp.float32),
        pltpu.SemaphoreType.DMA((NBUF,)),
        pltpu.SemaphoreType.DMA((NBUF,)),
    ],
    compiler_params=pltpu.CompilerParams(use_tc_tiling_on_sc=False),
)
def _embed_kernel(xa_hbm, xb_hbm, tok_hbm, pos_hbm, out_hbm,
                  xfA, xfB, idxA, idxB, rows, pos_v, sem_g, sem_o):
    wid = lax.axis_index("s") * NUM_CORES + lax.axis_index("c")
    sbase = wid * SEQ_W

    pltpu.sync_copy(xa_hbm.at[pl.ds(sbase, SEQ_W)], xfA)
    pltpu.sync_copy(xb_hbm.at[pl.ds(sbase, SEQ_W)], xfB)
    pltpu.sync_copy(pos_hbm, pos_v)

    @plsc.parallel_loop(0, SEQ_W, unroll=2)
    def _cvt(s):
        for c in range(HALF0 // 16):
            sl = pl.ds(c * 16, 16)
            idxA[s, sl] = lax.bitcast_convert_type(xfA[s, sl], jnp.int32)
        for c in range(XB_W // 16):
            sl = pl.ds(c * 16, 16)
            idxB[s, sl] = lax.bitcast_convert_type(xfB[s, sl], jnp.int32)

    def halves(k, b):
        h = b & 1
        off = HALF0 * h
        n = HALF1 if h else HALF0
        return k >> 1, off, n

    def idx_ref(s, b):
        if b & 1:
            return idxB.at[s, pl.ds(HALF0 - XB_OFF, HALF1)]
        return idxA.at[s, pl.ds(0, HALF0)]

    def g_issue(k, b):
        s, _, n = halves(k, b)
        pltpu.async_copy(
            tok_hbm.at[idx_ref(s, b)], rows.at[b, pl.ds(0, n)], sem_g.at[b])

    def g_wait(k, b):
        s, _, n = halves(k, b)
        pltpu.make_async_copy(
            tok_hbm.at[idx_ref(s, b)], rows.at[b, pl.ds(0, n)],
            sem_g.at[b]).wait()

    def o_copy(k, b):
        s, off, n = halves(k, b)
        return pltpu.make_async_copy(
            rows.at[b, pl.ds(0, n)],
            out_hbm.at[sbase + s, pl.ds(off, n), pl.ds(0, D)],
            sem_o.at[b])

    def chunk_step(k, b, issue_next, out_wait):
        g_wait(k, b)
        b2 = (b + LOOKAHEAD) % NBUF
        if out_wait:
            o_copy(k - (NBUF - LOOKAHEAD), b2).wait()
        if issue_next:
            g_issue(k + LOOKAHEAD, b2)
        _, off, n = halves(k, b)
        rows_b = rows.at[b]

        @plsc.parallel_loop(0, n, unroll=4)
        def _row(r):
            for c in range(D // 16):
                sl = pl.ds(c * 16, 16)
                rows_b[r, sl] = rows_b[r, sl] + pos_v[off + r, sl]

        o_copy(k, b).start()

    for j in range(LOOKAHEAD):
        g_issue(j, j)
    for k in range(NBUF):
        chunk_step(k, k, True, k >= NBUF - LOOKAHEAD)

    @pl.loop(1, NCHUNK // NBUF - 1)
    def _group(g):
        k0 = g * NBUF
        for b in range(NBUF):
            chunk_step(k0 + b, b, True, True)

    for k in range(NCHUNK - NBUF, NCHUNK):
        chunk_step(k, k % NBUF, k + LOOKAHEAD < NCHUNK, True)
    for k in range(NCHUNK - NBUF + LOOKAHEAD, NCHUNK):
        o_copy(k, k % NBUF).wait()


def kernel(x, token_table, position_table):
    xf = lax.bitcast_convert_type(x, jnp.float32)
    out = _embed_kernel(xf[:, :HALF0], xf[:, XB_OFF:], token_table,
                        position_table)
    return out[:, :, :D]


# speedup vs baseline: 1.0236x; 1.0223x over previous
import functools

import jax
import jax.numpy as jnp
from jax import lax
from jax.experimental import pallas as pl
from jax.experimental.pallas import tpu as pltpu
from jax.experimental.pallas import tpu_sc as plsc

B = 1024
L = 200
D = 64
DPAD = 128
VOCAB = 100000

NUM_CORES = 2
NUM_SUBCORES = 16
NW = NUM_CORES * NUM_SUBCORES
SEQ_W = B // NW
HALF0 = 128
HALF1 = L - HALF0
XB_W = 80
XB_OFF = L - XB_W
PAIR = 2
NCHUNK = 2 * SEQ_W // PAIR
NBUF = 4
LOOKAHEAD = 2

_mesh = plsc.VectorSubcoreMesh(core_axis_name="c", subcore_axis_name="s")


@functools.partial(
    pl.kernel,
    out_type=jax.ShapeDtypeStruct((B, L, DPAD), jnp.float32),
    mesh=_mesh,
    scratch_types=[
        pltpu.VMEM((SEQ_W, HALF0), jnp.float32),
        pltpu.VMEM((SEQ_W, XB_W), jnp.float32),
        pltpu.VMEM((SEQ_W, HALF0), jnp.int32),
        pltpu.VMEM((SEQ_W, XB_W), jnp.int32),
        pltpu.VMEM((NBUF, PAIR, HALF0, D), jnp.float32),
        pltpu.VMEM((L, D), jnp.float32),
        pltpu.SemaphoreType.DMA((NBUF,)),
        pltpu.SemaphoreType.DMA((NBUF,)),
    ],
    compiler_params=pltpu.CompilerParams(use_tc_tiling_on_sc=False),
)
def _embed_kernel(xa_hbm, xb_hbm, tok_hbm, pos_hbm, out_hbm,
                  xfA, xfB, idxA, idxB, rows, pos_v, sem_g, sem_o):
    wid = lax.axis_index("s") * NUM_CORES + lax.axis_index("c")
    sbase = wid * SEQ_W

    pltpu.sync_copy(xa_hbm.at[pl.ds(sbase, SEQ_W)], xfA)
    pltpu.sync_copy(xb_hbm.at[pl.ds(sbase, SEQ_W)], xfB)
    pltpu.sync_copy(pos_hbm, pos_v)

    @plsc.parallel_loop(0, SEQ_W, unroll=2)
    def _cvt(s):
        for c in range(HALF0 // 16):
            sl = pl.ds(c * 16, 16)
            idxA[s, sl] = lax.bitcast_convert_type(xfA[s, sl], jnp.int32)
        for c in range(XB_W // 16):
            sl = pl.ds(c * 16, 16)
            idxB[s, sl] = lax.bitcast_convert_type(xfB[s, sl], jnp.int32)

    def halves(k, b):
        h = b & 1
        off = HALF0 * h
        n = HALF1 if h else HALF0
        return (k >> 1) * PAIR, off, n

    def idx_ref(s, b):
        if b & 1:
            return idxB.at[s, pl.ds(HALF0 - XB_OFF, HALF1)]
        return idxA.at[s, pl.ds(0, HALF0)]

    def g_copies(k, b):
        s, _, n = halves(k, b)
        return [pltpu.make_async_copy(
                    tok_hbm.at[idx_ref(s + j, b)],
                    rows.at[b, j, pl.ds(0, n)], sem_g.at[b])
                for j in range(PAIR)]

    def o_copies(k, b):
        s, off, n = halves(k, b)
        return [pltpu.make_async_copy(
                    rows.at[b, j, pl.ds(0, n)],
                    out_hbm.at[sbase + s + j, pl.ds(off, n), pl.ds(0, D)],
                    sem_o.at[b])
                for j in range(PAIR)]

    def chunk_step(k, b, issue_next, out_wait):
        for cp in g_copies(k, b):
            cp.wait()
        b2 = (b + LOOKAHEAD) % NBUF
        if out_wait:
            for cp in o_copies(k - (NBUF - LOOKAHEAD), b2):
                cp.wait()
        if issue_next:
            for cp in g_copies(k + LOOKAHEAD, b2):
                cp.start()
        _, off, n = halves(k, b)
        rows_b = rows.at[b]

        @plsc.parallel_loop(0, n, unroll=4)
        def _row(r):
            for c in range(D // 16):
                sl = pl.ds(c * 16, 16)
                pv = pos_v[off + r, sl]
                for j in range(PAIR):
                    rows_b[j, r, sl] = rows_b[j, r, sl] + pv

        for cp in o_copies(k, b):
            cp.start()

    for j in range(LOOKAHEAD):
        for cp in g_copies(j, j):
            cp.start()
    for k in range(NBUF):
        chunk_step(k, k, True, k >= NBUF - LOOKAHEAD)

    @pl.loop(1, NCHUNK // NBUF - 1)
    def _group(g):
        k0 = g * NBUF
        for b in range(NBUF):
            chunk_step(k0 + b, b, True, True)

    for k in range(NCHUNK - NBUF, NCHUNK):
        chunk_step(k, k % NBUF, k + LOOKAHEAD < NCHUNK, True)
    for k in range(NCHUNK - NBUF + LOOKAHEAD, NCHUNK):
        for cp in o_copies(k, k % NBUF):
            cp.wait()


def kernel(x, token_table, position_table):
    xf = lax.bitcast_convert_type(x, jnp.float32)
    out = _embed_kernel(xf[:, :HALF0], xf[:, XB_OFF:], token_table,
                        position_table)
    return out[:, :, :D]
